# baseline (speedup 1.0000x reference)
import functools

import jax
import jax.numpy as jnp
from jax import lax
from jax.experimental import pallas as pl
from jax.experimental.pallas import tpu as pltpu
from jax.experimental.pallas import tpu_sc as plsc

NCLS = 80
NPROP = 20000
NGT = 100
GPAD = 112
L = 16
GB = 4


def _body(nc, ns, per_w, px1, py1, px2, py2,
          gb1, gb2, gb3, gb4, g1, g2, g3, g4, gcls,
          ovals, oidxs, ocls, ob1, ob2, ob3, ob4, ooh,
          pv1, pv2, pv3, pv4, vb1, vb2, vb3, vb4,
          vg1, vg2, vg3, vg4, vcls,
          sv, si, sc, sb1, sb2, sb3, sb4, voh, sem):
    wid = lax.axis_index("s") * nc + lax.axis_index("c")
    base = wid * per_w

    sl = pl.ds(base, per_w)
    cps = [
        pltpu.async_copy(px1.at[sl], pv1, sem),
        pltpu.async_copy(py1.at[sl], pv2, sem),
        pltpu.async_copy(px2.at[sl], pv3, sem),
        pltpu.async_copy(py2.at[sl], pv4, sem),
        pltpu.async_copy(gb1, vb1, sem),
        pltpu.async_copy(gb2, vb2, sem),
        pltpu.async_copy(gb3, vb3, sem),
        pltpu.async_copy(gb4, vb4, sem),
        pltpu.async_copy(g1, vg1, sem),
        pltpu.async_copy(g2, vg2, sem),
        pltpu.async_copy(g3, vg3, sem),
        pltpu.async_copy(g4, vg4, sem),
        pltpu.async_copy(gcls, vcls, sem),
    ]
    for c in cps:
        c.wait()

    groups = per_w // L
    g0 = 0
    while g0 < groups:
        gcount = min(GB, groups - g0)
        offs = [(g0 + j) * L for j in range(gcount)]
        x1s = [pv1[pl.ds(o, L)] for o in offs]
        y1s = [pv2[pl.ds(o, L)] for o in offs]
        x2s = [pv3[pl.ds(o, L)] for o in offs]
        y2s = [pv4[pl.ds(o, L)] for o in offs]
        pas = [(x2s[j] - x1s[j]) * (y2s[j] - y1s[j]) for j in range(gcount)]

        zf = jnp.zeros((L,), jnp.float32)
        onef = jnp.ones((L,), jnp.float32)
        zi = jnp.zeros((L,), jnp.int32)
        init = (tuple(zf for _ in range(gcount)),
                tuple(onef for _ in range(gcount)),
                tuple(zi for _ in range(gcount)))

        def mbody(m, carry, x1s=x1s, y1s=y1s, x2s=x2s, y2s=y2s, pas=pas,
                  gcount=gcount):
            bis, bus, bids = carry
            mo = m * L
            gx1 = vb1[pl.ds(mo, L)]
            gy1 = vb2[pl.ds(mo, L)]
            gx2 = vb3[pl.ds(mo, L)]
            gy2 = vb4[pl.ds(mo, L)]
            ga = (gx2 - gx1) * (gy2 - gy1)
            midx = jnp.full((L,), m, jnp.int32)
            nbi, nbu, nbd = [], [], []
            for j in range(gcount):
                ltx = jnp.maximum(gx1, x1s[j])
                lty = jnp.maximum(gy1, y1s[j])
                rbx = jnp.minimum(gx2, x2s[j])
                rby = jnp.minimum(gy2, y2s[j])
                w = jnp.maximum(rbx - ltx, 0.0)
                h = jnp.maximum(rby - lty, 0.0)
                inter = w * h
                union = ga + pas[j] - inter
                upd = inter * bus[j] > bis[j] * union
                nbi.append(jnp.where(upd, inter, bis[j]))
                nbu.append(jnp.where(upd, union, bus[j]))
                nbd.append(jnp.where(upd, midx, bids[j]))
            return (tuple(nbi), tuple(nbu), tuple(nbd))

        bis, bus, bids = lax.fori_loop(0, NGT, mbody, init)

        for j in range(gcount):
            o = offs[j]
            vals = bis[j] / bus[j]
            fg = vals >= 0.5
            idx = bids[j]
            cls = plsc.load_gather(vcls, [idx])
            cls = jnp.where(fg, cls, NCLS)
            sv[pl.ds(o, L)] = vals
            si[pl.ds(o, L)] = idx
            sc[pl.ds(o, L)] = cls
            sb1[pl.ds(o, L)] = plsc.load_gather(vg1, [idx])
            sb2[pl.ds(o, L)] = plsc.load_gather(vg2, [idx])
            sb3[pl.ds(o, L)] = plsc.load_gather(vg3, [idx])
            sb4[pl.ds(o, L)] = plsc.load_gather(vg4, [idx])
        g0 += gcount

    @pl.when(wid == 0)
    def _():
        zf16 = jnp.zeros((L,), jnp.float32)
        for c in range(96 // L):
            voh[pl.ds(c * L, L)] = zf16
        ones = jnp.ones((L,), jnp.float32)
        for c in range(GPAD // L):
            ids = vcls[pl.ds(c * L, L)]
            plsc.store_scatter(voh, [ids], ones)
        pltpu.sync_copy(voh, ooh)

    outs = [
        pltpu.async_copy(sv, ovals.at[sl], sem),
        pltpu.async_copy(si, oidxs.at[sl], sem),
        pltpu.async_copy(sc, ocls.at[sl], sem),
        pltpu.async_copy(sb1, ob1.at[sl], sem),
        pltpu.async_copy(sb2, ob2.at[sl], sem),
        pltpu.async_copy(sb3, ob3.at[sl], sem),
        pltpu.async_copy(sb4, ob4.at[sl], sem),
    ]
    for c in outs:
        c.wait()


def kernel(proposal_boxes, gt_boxes, gt_classes):
    try:
        info = plsc.get_sparse_core_info()
        nc, ns = 1, info.num_subcores
    except Exception:
        nc, ns = 1, 16
    nw = nc * ns
    per_w = (-(-NPROP // nw) + L - 1) // L * L
    npad = per_w * nw

    pb = jnp.pad(proposal_boxes, ((0, npad - NPROP), (0, 0)))
    px1, py1, px2, py2 = (pb[:, 0], pb[:, 1], pb[:, 2], pb[:, 3])
    gbp = jnp.pad(gt_boxes, ((0, GPAD - NGT), (0, 0)))
    g1, g2, g3, g4 = (gbp[:, 0], gbp[:, 1], gbp[:, 2], gbp[:, 3])
    gb1, gb2, gb3, gb4 = (jnp.repeat(g, L) for g in (g1, g2, g3, g4))
    gcls = jnp.pad(gt_classes.astype(jnp.int32), (0, GPAD - NGT),
                   constant_values=NCLS)

    mesh = plsc.VectorSubcoreMesh(core_axis_name="c", subcore_axis_name="s",
                                  num_cores=nc, num_subcores=ns)
    f32, i32 = jnp.float32, jnp.int32
    out_type = (
        jax.ShapeDtypeStruct((npad,), f32),
        jax.ShapeDtypeStruct((npad,), i32),
        jax.ShapeDtypeStruct((npad,), i32),
        jax.ShapeDtypeStruct((npad,), f32),
        jax.ShapeDtypeStruct((npad,), f32),
        jax.ShapeDtypeStruct((npad,), f32),
        jax.ShapeDtypeStruct((npad,), f32),
        jax.ShapeDtypeStruct((96,), f32),
    )
    scratch = [
        pltpu.VMEM((per_w,), f32), pltpu.VMEM((per_w,), f32),
        pltpu.VMEM((per_w,), f32), pltpu.VMEM((per_w,), f32),
        pltpu.VMEM((GPAD * L,), f32), pltpu.VMEM((GPAD * L,), f32),
        pltpu.VMEM((GPAD * L,), f32), pltpu.VMEM((GPAD * L,), f32),
        pltpu.VMEM((GPAD,), f32), pltpu.VMEM((GPAD,), f32),
        pltpu.VMEM((GPAD,), f32), pltpu.VMEM((GPAD,), f32),
        pltpu.VMEM((GPAD,), i32),
        pltpu.VMEM((per_w,), f32), pltpu.VMEM((per_w,), i32),
        pltpu.VMEM((per_w,), i32),
        pltpu.VMEM((per_w,), f32), pltpu.VMEM((per_w,), f32),
        pltpu.VMEM((per_w,), f32), pltpu.VMEM((per_w,), f32),
        pltpu.VMEM((96,), f32),
        pltpu.SemaphoreType.DMA,
    ]
    run = pl.kernel(functools.partial(_body, nc, ns, per_w),
                    out_type=out_type, mesh=mesh, scratch_types=scratch,
                    compiler_params=pltpu.CompilerParams(
                        needs_layout_passes=False))
    vals, idxs, cls, b1, b2, b3, b4, oh = run(
        px1, py1, px2, py2, gb1, gb2, gb3, gb4, g1, g2, g3, g4, gcls)
    boxes = jnp.stack([b1[:NPROP], b2[:NPROP], b3[:NPROP], b4[:NPROP]],
                      axis=1)
    return (vals[:NPROP], idxs[:NPROP], cls[:NPROP], boxes,
            oh[:NCLS + 1])


# speedup vs baseline: 1.3228x; 1.3228x over previous
import functools

import jax
import jax.numpy as jnp
from jax import lax
from jax.experimental import pallas as pl
from jax.experimental.pallas import tpu as pltpu
from jax.experimental.pallas import tpu_sc as plsc

NCLS = 80
NPROP = 20000
NGT = 100
GPAD = 112
L = 16
GB = 4


def _body(nc, ns, per_w, px1, py1, px2, py2,
          gb1, gb2, gb3, gb4, g1, g2, g3, g4, gcls,
          ovals, oidxs, ocls, ob1, ob2, ob3, ob4, ooh,
          pv1, pv2, pv3, pv4, vb1, vb2, vb3, vb4,
          vg1, vg2, vg3, vg4, vcls,
          sv, si, sc, sb1, sb2, sb3, sb4, voh, sem):
    wid = lax.axis_index("s") * nc + lax.axis_index("c")
    base = wid * per_w

    sl = pl.ds(base, per_w)
    cps = [
        pltpu.async_copy(px1.at[sl], pv1, sem),
        pltpu.async_copy(py1.at[sl], pv2, sem),
        pltpu.async_copy(px2.at[sl], pv3, sem),
        pltpu.async_copy(py2.at[sl], pv4, sem),
        pltpu.async_copy(gb1, vb1, sem),
        pltpu.async_copy(gb2, vb2, sem),
        pltpu.async_copy(gb3, vb3, sem),
        pltpu.async_copy(gb4, vb4, sem),
        pltpu.async_copy(g1, vg1, sem),
        pltpu.async_copy(g2, vg2, sem),
        pltpu.async_copy(g3, vg3, sem),
        pltpu.async_copy(g4, vg4, sem),
        pltpu.async_copy(gcls, vcls, sem),
    ]
    for c in cps:
        c.wait()

    nblk = per_w // (GB * L)

    def bbody(b, _):
        off = b * (GB * L)
        offs = [off + j * L for j in range(GB)]
        x1s = [pv1[pl.ds(o, L)] for o in offs]
        y1s = [pv2[pl.ds(o, L)] for o in offs]
        x2s = [pv3[pl.ds(o, L)] for o in offs]
        y2s = [pv4[pl.ds(o, L)] for o in offs]
        pas = [(x2s[j] - x1s[j]) * (y2s[j] - y1s[j]) for j in range(GB)]

        zf = jnp.zeros((L,), jnp.float32)
        onef = jnp.ones((L,), jnp.float32)
        zi = jnp.zeros((L,), jnp.int32)
        init = (tuple(zf for _ in range(GB)),
                tuple(onef for _ in range(GB)),
                tuple(zi for _ in range(GB)))

        def mbody(m, carry):
            bis, bus, bids = carry
            mo = m * L
            gx1 = vb1[pl.ds(mo, L)]
            gy1 = vb2[pl.ds(mo, L)]
            gx2 = vb3[pl.ds(mo, L)]
            gy2 = vb4[pl.ds(mo, L)]
            ga = (gx2 - gx1) * (gy2 - gy1)
            midx = jnp.full((L,), m, jnp.int32)
            nbi, nbu, nbd = [], [], []
            for j in range(GB):
                ltx = jnp.maximum(gx1, x1s[j])
                lty = jnp.maximum(gy1, y1s[j])
                rbx = jnp.minimum(gx2, x2s[j])
                rby = jnp.minimum(gy2, y2s[j])
                w = jnp.maximum(rbx - ltx, 0.0)
                h = jnp.maximum(rby - lty, 0.0)
                inter = w * h
                union = ga + pas[j] - inter
                upd = inter * bus[j] > bis[j] * union
                nbi.append(jnp.where(upd, inter, bis[j]))
                nbu.append(jnp.where(upd, union, bus[j]))
                nbd.append(jnp.where(upd, midx, bids[j]))
            return (tuple(nbi), tuple(nbu), tuple(nbd))

        bis, bus, bids = lax.fori_loop(0, NGT, mbody, init)

        for j in range(GB):
            o = offs[j]
            vals = bis[j] / bus[j]
            fg = vals >= 0.5
            idx = bids[j]
            cls = plsc.load_gather(vcls, [idx])
            cls = jnp.where(fg, cls, NCLS)
            sv[pl.ds(o, L)] = vals
            si[pl.ds(o, L)] = idx
            sc[pl.ds(o, L)] = cls
            sb1[pl.ds(o, L)] = plsc.load_gather(vg1, [idx])
            sb2[pl.ds(o, L)] = plsc.load_gather(vg2, [idx])
            sb3[pl.ds(o, L)] = plsc.load_gather(vg3, [idx])
            sb4[pl.ds(o, L)] = plsc.load_gather(vg4, [idx])
        return 0

    lax.fori_loop(0, nblk, bbody, 0)

    @pl.when(wid == 0)
    def _():
        zf16 = jnp.zeros((L,), jnp.float32)
        for c in range(96 // L):
            voh[pl.ds(c * L, L)] = zf16
        ones = jnp.ones((L,), jnp.float32)
        for c in range(GPAD // L):
            ids = vcls[pl.ds(c * L, L)]
            plsc.store_scatter(voh, [ids], ones)
        pltpu.sync_copy(voh, ooh)

    outs = [
        pltpu.async_copy(sv, ovals.at[sl], sem),
        pltpu.async_copy(si, oidxs.at[sl], sem),
        pltpu.async_copy(sc, ocls.at[sl], sem),
        pltpu.async_copy(sb1, ob1.at[sl], sem),
        pltpu.async_copy(sb2, ob2.at[sl], sem),
        pltpu.async_copy(sb3, ob3.at[sl], sem),
        pltpu.async_copy(sb4, ob4.at[sl], sem),
    ]
    for c in outs:
        c.wait()


def kernel(proposal_boxes, gt_boxes, gt_classes):
    try:
        info = plsc.get_sparse_core_info()
        nc, ns = info.num_cores, info.num_subcores
    except Exception:
        nc, ns = 2, 16
    nw = nc * ns
    blk = GB * L
    per_w = (-(-NPROP // nw) + blk - 1) // blk * blk
    npad = per_w * nw

    pb = jnp.pad(proposal_boxes, ((0, npad - NPROP), (0, 0)))
    px1, py1, px2, py2 = (pb[:, 0], pb[:, 1], pb[:, 2], pb[:, 3])
    gbp = jnp.pad(gt_boxes, ((0, GPAD - NGT), (0, 0)))
    g1, g2, g3, g4 = (gbp[:, 0], gbp[:, 1], gbp[:, 2], gbp[:, 3])
    gb1, gb2, gb3, gb4 = (jnp.repeat(g, L) for g in (g1, g2, g3, g4))
    gcls = jnp.pad(gt_classes.astype(jnp.int32), (0, GPAD - NGT),
                   constant_values=NCLS)

    mesh = plsc.VectorSubcoreMesh(core_axis_name="c", subcore_axis_name="s",
                                  num_cores=nc, num_subcores=ns)
    f32, i32 = jnp.float32, jnp.int32
    out_type = (
        jax.ShapeDtypeStruct((npad,), f32),
        jax.ShapeDtypeStruct((npad,), i32),
        jax.ShapeDtypeStruct((npad,), i32),
        jax.ShapeDtypeStruct((npad,), f32),
        jax.ShapeDtypeStruct((npad,), f32),
        jax.ShapeDtypeStruct((npad,), f32),
        jax.ShapeDtypeStruct((npad,), f32),
        jax.ShapeDtypeStruct((96,), f32),
    )
    scratch = [
        pltpu.VMEM((per_w,), f32), pltpu.VMEM((per_w,), f32),
        pltpu.VMEM((per_w,), f32), pltpu.VMEM((per_w,), f32),
        pltpu.VMEM((GPAD * L,), f32), pltpu.VMEM((GPAD * L,), f32),
        pltpu.VMEM((GPAD * L,), f32), pltpu.VMEM((GPAD * L,), f32),
        pltpu.VMEM((GPAD,), f32), pltpu.VMEM((GPAD,), f32),
        pltpu.VMEM((GPAD,), f32), pltpu.VMEM((GPAD,), f32),
        pltpu.VMEM((GPAD,), i32),
        pltpu.VMEM((per_w,), f32), pltpu.VMEM((per_w,), i32),
        pltpu.VMEM((per_w,), i32),
        pltpu.VMEM((per_w,), f32), pltpu.VMEM((per_w,), f32),
        pltpu.VMEM((per_w,), f32), pltpu.VMEM((per_w,), f32),
        pltpu.VMEM((96,), f32),
        pltpu.SemaphoreType.DMA,
    ]
    run = pl.kernel(functools.partial(_body, nc, ns, per_w),
                    out_type=out_type, mesh=mesh, scratch_types=scratch,
                    compiler_params=pltpu.CompilerParams(
                        needs_layout_passes=False))
    vals, idxs, cls, b1, b2, b3, b4, oh = run(
        px1, py1, px2, py2, gb1, gb2, gb3, gb4, g1, g2, g3, g4, gcls)
    boxes = jnp.stack([b1[:NPROP], b2[:NPROP], b3[:NPROP], b4[:NPROP]],
                      axis=1)
    return (vals[:NPROP], idxs[:NPROP], cls[:NPROP], boxes,
            oh[:NCLS + 1])
